# Initial kernel scaffold; baseline (speedup 1.0000x reference)
import jax
import jax.numpy as jnp
from jax.experimental import pallas as pl


def kernel(problems, rels, targets, labels, prob_embed, ord_embed, rel_embed):
    raise NotImplementedError("write your pallas kernel here")


# speedup vs baseline: 3.4210x; 3.4210x over previous
import jax
import jax.numpy as jnp
from jax import lax
from jax.experimental import pallas as pl
from jax.experimental.pallas import tpu as pltpu
from jax.experimental.pallas import tpu_sc as plsc

NUM_RELATION_TYPES = 3
EMBED_SIZE = 300
BATCH = 16384
GROUP = 4

NC = 2
NS = 16
NW = NC * NS
L = 16
BPW = BATCH // NW
CHUNK = 128
NCHUNK = BPW // CHUNK
NFULL = EMBED_SIZE // L


def _body(problems_hbm, rels_hbm, targets_hbm, prob_hbm, ord_hbm, rel_hbm,
          out_hbm, pidx_v, tidx_v, ridx_v, rel_v, p_rows, t_rows,
          partial_v, sem):
    wid = lax.axis_index("s") * NC + lax.axis_index("c")
    base = wid * BPW

    pltpu.sync_copy(problems_hbm.at[pl.ds(base, BPW)], pidx_v)
    pltpu.sync_copy(targets_hbm.at[pl.ds(base, BPW)], tidx_v)
    pltpu.sync_copy(rels_hbm.at[pl.ds(base, BPW)], ridx_v)
    pltpu.sync_copy(rel_hbm, rel_v)

    iota = lax.iota(jnp.int32, L)
    tail_mask = iota >= (L - (EMBED_SIZE - NFULL * L))
    tail_off = EMBED_SIZE - L

    def fetch(c):
        coff = pl.multiple_of(c * CHUNK, CHUNK)

        def issue(g, acc):
            goff = pl.multiple_of(g * L, L)
            pidx = pidx_v[pl.ds(coff + goff, L)]
            tidx = tidx_v[pl.ds(coff + goff, L)]
            for k in range(L):
                pltpu.async_copy(prob_hbm.at[pidx[k]],
                                 p_rows.at[goff + k], sem)
                pltpu.async_copy(ord_hbm.at[tidx[k]],
                                 t_rows.at[goff + k], sem)
            return acc

        lax.fori_loop(0, CHUNK // L, issue, jnp.int32(0))

    def drain():
        pltpu.make_async_copy(prob_hbm.at[pl.ds(0, CHUNK)], p_rows, sem).wait()
        pltpu.make_async_copy(ord_hbm.at[pl.ds(0, CHUNK)], t_rows, sem).wait()

    def chunk_body(c, lsum):
        coff = pl.multiple_of(c * CHUNK, CHUNK)
        fetch(c)
        drain()

        def group_body(g, lsum):
            goff = pl.multiple_of(g * L, L)
            rid_vec = ridx_v[pl.ds(coff + goff, L)]
            cur = jnp.float32(0.0)
            for k in range(L):
                i = goff + k
                rid = rid_vec[k]
                acc = jnp.zeros((L,), jnp.float32)
                for j in range(NFULL):
                    pj = p_rows[i, pl.ds(j * L, L)]
                    tj = t_rows[i, pl.ds(j * L, L)]
                    rj = rel_v[rid, pl.ds(j * L, L)]
                    acc = acc + pj * tj * rj
                pj = p_rows[i, pl.ds(tail_off, L)]
                tj = t_rows[i, pl.ds(tail_off, L)]
                rj = rel_v[rid, pl.ds(tail_off, L)]
                acc = acc + jnp.where(tail_mask, pj * tj * rj, 0.0)
                for sh in (1, 2, 4, 8):
                    acc = acc + jnp.take(acc, iota ^ sh)
                s = acc[0]
                if k % GROUP == 0:
                    cur = s
                else:
                    lsum = lsum + jnp.maximum(s - cur + 1.0, 0.0)
            return lsum

        return lax.fori_loop(0, CHUNK // L, group_body, lsum)

    lsum = lax.fori_loop(0, NCHUNK, chunk_body, jnp.float32(0.0))

    partial_v[...] = jnp.where(iota == 0, lsum * (GROUP / BATCH), 0.0)
    pltpu.sync_copy(partial_v, out_hbm.at[wid])


_mesh = plsc.VectorSubcoreMesh(core_axis_name="c", subcore_axis_name="s")

_sc_call = pl.kernel(
    _body,
    out_type=jax.ShapeDtypeStruct((NW, L), jnp.float32),
    mesh=_mesh,
    scratch_types=[
        pltpu.VMEM((BPW,), jnp.int32),
        pltpu.VMEM((BPW,), jnp.int32),
        pltpu.VMEM((BPW,), jnp.int32),
        pltpu.VMEM((NUM_RELATION_TYPES, EMBED_SIZE), jnp.float32),
        pltpu.VMEM((CHUNK, EMBED_SIZE), jnp.float32),
        pltpu.VMEM((CHUNK, EMBED_SIZE), jnp.float32),
        pltpu.VMEM((L,), jnp.float32),
        pltpu.SemaphoreType.DMA,
    ],
)


@jax.jit
def kernel(problems, rels, targets, labels, prob_embed, ord_embed, rel_embed):
    del labels
    out = _sc_call(problems.astype(jnp.int32), rels.astype(jnp.int32),
                   targets.astype(jnp.int32), prob_embed, ord_embed,
                   rel_embed)
    return jnp.sum(out)
